# baseline (speedup 1.0000x reference)
import functools

import jax
import jax.numpy as jnp
from jax import lax
from jax.experimental import pallas as pl
from jax.experimental.pallas import tpu as pltpu
from jax.experimental.pallas import tpu_sc as plsc

B, T, D = 4096, 77, 768
NW = 32
BPW = B // NW
C0 = 40
C1 = 37
TA = 32
NG = 2 * BPW
GB = 8
GSZ = 2 * GB
NGRP = BPW // GB
NDV = D // 16


def _body(x2, tok, pos, out, pos_all, idx3, bufA, bufB,
          isem, gsA, gsB, ssA, ssB):
    wid = lax.axis_index("s") * 2 + lax.axis_index("c")
    b0 = wid * BPW

    pltpu.sync_copy(pos, pos_all)

    def idx_load(grp):
        sel = lax.rem(grp, 3)
        return pltpu.make_async_copy(
            x2.at[pl.ds(b0 + grp * GB, GB)], idx3.at[sel], isem.at[sel])

    def gcopy(g, buf, sem, s, c):
        sel = lax.rem(g // GSZ, 3)
        m = lax.rem(g // 2, GB)
        return pltpu.make_async_copy(
            tok.at[idx3.at[sel, m, pl.ds(s, c)]], buf.at[pl.ds(0, c)], sem)

    def scatter_pieces(g, buf, s):
        b = b0 + g // 2
        if s == 0:
            yield buf, out.at[b, pl.ds(0, C0)]
        else:
            yield buf.at[pl.ds(0, TA)], out.at[b, pl.ds(C0, TA)]
            for i in range(C1 - TA):
                yield (buf.at[pl.ds(TA + i, 1)],
                       out.at[b, pl.ds(C0 + TA + i, 1)])

    def scatter(g, buf, sem, s):
        for src, dst in scatter_pieces(g, buf, s):
            pltpu.async_copy(src, dst, sem)

    def scatter_wait(g, buf, sem, s):
        for src, dst in scatter_pieces(g, buf, s):
            pltpu.make_async_copy(src, dst, sem).wait()

    def add_pos(buf, s, c):
        def r_body(r, _):
            for j in range(NDV):
                pv = pos_all[s + r, pl.ds(j * 16, 16)]
                plsc.addupdate(buf.at[r, pl.ds(j * 16, 16)], pv)
            return 0

        lax.fori_loop(0, c, r_body, 0)

    bufs = ((bufA, gsA, ssA, 0, C0), (bufB, gsB, ssB, C0, C1))

    idx_load(0).start()
    idx_load(1).start()
    idx_load(0).wait()
    gcopy(0, bufA, gsA, 0, C0).start()

    def g2_body(g2, _):
        for bpar in range(2):
            g = g2 * 2 + bpar
            cur_buf, cur_g, cur_s, s, c = bufs[bpar]
            nxt_buf, nxt_g, nxt_s, sn, cn = bufs[1 - bpar]

            if bpar == 0:
                @pl.when(lax.rem(g, GSZ) == 0)
                def _():
                    grp = g // GSZ

                    @pl.when(grp >= 1)
                    def _():
                        idx_load(grp).wait()

                    @pl.when(grp + 2 < NGRP)
                    def _():
                        idx_load(grp + 2).start()

            @pl.when(g >= 1)
            def _():
                scatter_wait(g - 1, nxt_buf, nxt_s, sn)

            @pl.when(g + 1 < NG)
            def _():
                gcopy(g + 1, nxt_buf, nxt_g, sn, cn).start()

            gcopy(g, cur_buf, cur_g, s, c).wait()
            add_pos(cur_buf, s, c)
            scatter(g, cur_buf, cur_s, s)
        return 0

    lax.fori_loop(0, NG // 2, g2_body, 0)
    scatter_wait(NG - 1, bufB, ssB, C0)


@jax.jit
def kernel(x, tok_embed, pos_embed):
    x2 = x.astype(jnp.int32)
    mesh = plsc.VectorSubcoreMesh(core_axis_name="c", subcore_axis_name="s")
    k = functools.partial(
        pl.kernel,
        mesh=mesh,
        out_type=jax.ShapeDtypeStruct((B, T, D), jnp.float32),
        scratch_types=[
            pltpu.VMEM((T, D), jnp.float32),
            pltpu.VMEM((3, GB, T), jnp.int32),
            pltpu.VMEM((C0, D), jnp.float32),
            pltpu.VMEM((C1, D), jnp.float32),
            pltpu.SemaphoreType.DMA((3,)),
            pltpu.SemaphoreType.DMA,
            pltpu.SemaphoreType.DMA,
            pltpu.SemaphoreType.DMA,
            pltpu.SemaphoreType.DMA,
        ],
    )(_body)
    return k(x2, tok_embed, pos_embed)


# speedup vs baseline: 1.0469x; 1.0469x over previous
import functools

import jax
import jax.numpy as jnp
from jax import lax
from jax.experimental import pallas as pl
from jax.experimental.pallas import tpu as pltpu
from jax.experimental.pallas import tpu_sc as plsc

B, T, D = 4096, 77, 768
NW = 32
BPW = B // NW
TP = 80
CPB = 3
CS = (0, 32, 64)
CC = (32, 32, 16)
NG = CPB * BPW
GB = 8
GSZ = CPB * GB
NGRP = BPW // GB
GW = GB * TP
NT = 5
NDV = D // 16


def _body(x2, tok, pos, out, pos_all, idxf, bufA, bufB, tb0, tb1,
          isem, gsA, gsB, ssA, ssB, ts0, ts1):
    wid = lax.axis_index("s") * 2 + lax.axis_index("c")
    b0 = wid * BPW
    bufs, gs, ss = (bufA, bufB), (gsA, gsB), (ssA, ssB)
    tbs, tss = (tb0, tb1), (ts0, ts1)

    pltpu.sync_copy(pos, pos_all)

    def idx_load(grp):
        sel = lax.rem(grp, 3)
        return pltpu.make_async_copy(
            x2.at[pl.ds((b0 + grp * GB) * TP, GW)],
            idxf.at[pl.ds(sel * GW, GW)], isem.at[sel])

    def gcopy(k, buf, sem, kind):
        sel = lax.rem(k // GSZ, 3)
        m = lax.rem(k // CPB, GB)
        return pltpu.make_async_copy(
            tok.at[idxf.at[pl.ds(sel * GW + m * TP + CS[kind], CC[kind])]],
            buf.at[0, pl.ds(0, CC[kind])], sem)

    def slab_copy(k, buf, sem, kind):
        b = b0 + k // CPB
        if kind < 2:
            return pltpu.make_async_copy(
                buf, out.at[pl.ds(b, 1), pl.ds(CS[kind], 32)], sem)
        return pltpu.make_async_copy(
            buf.at[pl.ds(0, 1), pl.ds(0, 8)],
            out.at[pl.ds(b, 1), pl.ds(64, 8)], sem)

    def tail_copy(bloc, tpar, i):
        return pltpu.make_async_copy(
            tbs[tpar].at[pl.ds(i, 1)],
            out.at[pl.ds(b0 + bloc, 1), pl.ds(72 + i, 1)], tss[tpar])

    def add_pos(buf, tbase, nrows):
        def r_body(r, _):
            for j in range(NDV):
                pv = pos_all[tbase + r, pl.ds(j * 16, 16)]
                plsc.addupdate(buf.at[0, r, pl.ds(j * 16, 16)], pv)
            return 0

        lax.fori_loop(0, nrows, r_body, 0)

    def tail_move_add(buf, tb):
        for i in range(NT):
            for j in range(NDV):
                tb[i, 0, pl.ds(j * 16, 16)] = (
                    buf[0, 8 + i, pl.ds(j * 16, 16)]
                    + pos_all[72 + i, pl.ds(j * 16, 16)])

    idx_load(0).start()
    idx_load(1).start()
    idx_load(0).wait()
    gcopy(0, bufA, gsA, 0).start()

    def blk_body(i2, _):
        for u in range(2 * CPB):
            k = i2 * (2 * CPB) + u
            kind = u % CPB
            par = u % 2
            tpar = u // CPB
            u_prev = (u + 2 * CPB - 1) % (2 * CPB)
            kind_prev = u_prev % CPB
            cur_buf, cur_g = bufs[par], gs[par]
            nxt_buf, nxt_g = bufs[1 - par], gs[1 - par]

            if u == 0:
                @pl.when(lax.rem(k, GSZ) == 0)
                def _():
                    grp = k // GSZ

                    @pl.when(grp + 1 < NGRP)
                    def _():
                        idx_load(grp + 1).wait()

                    @pl.when(grp + 2 < NGRP)
                    def _():
                        idx_load(grp + 2).start()

            @pl.when(k >= 1)
            def _():
                slab_copy(k - 1, nxt_buf, ss[1 - par], kind_prev).wait()

            @pl.when(k + 1 < NG)
            def _():
                gcopy(k + 1, nxt_buf, nxt_g, (u + 1) % CPB).start()

            gcopy(k, cur_buf, cur_g, kind).wait()
            if kind < 2:
                add_pos(cur_buf, CS[kind], 32)
                slab_copy(k, cur_buf, ss[par], kind).start()
            else:
                add_pos(cur_buf, 64, 8)

                @pl.when(k >= 2 * CPB)
                def _():
                    for i in range(NT):
                        tail_copy(k // CPB - 2, tpar, i).wait()

                tail_move_add(cur_buf, tbs[tpar])
                slab_copy(k, cur_buf, ss[par], 2).start()
                for i in range(NT):
                    tail_copy(k // CPB, tpar, i).start()
        return 0

    lax.fori_loop(0, NG // (2 * CPB), blk_body, 0)

    slab_copy(NG - 1, bufB, ssB, 2).wait()
    for tpar, bloc in ((0, BPW - 2), (1, BPW - 1)):
        for i in range(NT):
            tail_copy(bloc, tpar, i).wait()


@jax.jit
def kernel(x, tok_embed, pos_embed):
    x2 = jnp.pad(x.astype(jnp.int32), ((0, 0), (0, TP - T))).reshape(B * TP)
    mesh = plsc.VectorSubcoreMesh(core_axis_name="c", subcore_axis_name="s")
    k = functools.partial(
        pl.kernel,
        mesh=mesh,
        out_type=jax.ShapeDtypeStruct((B, T, D), jnp.float32),
        scratch_types=[
            pltpu.VMEM((T, D), jnp.float32),
            pltpu.VMEM((3 * GW,), jnp.int32),
            pltpu.VMEM((1, 32, D), jnp.float32),
            pltpu.VMEM((1, 32, D), jnp.float32),
            pltpu.VMEM((NT, 1, D), jnp.float32),
            pltpu.VMEM((NT, 1, D), jnp.float32),
            pltpu.SemaphoreType.DMA((3,)),
            pltpu.SemaphoreType.DMA,
            pltpu.SemaphoreType.DMA,
            pltpu.SemaphoreType.DMA,
            pltpu.SemaphoreType.DMA,
            pltpu.SemaphoreType.DMA,
            pltpu.SemaphoreType.DMA,
        ],
    )(_body)
    return k(x2, tok_embed, pos_embed)


# speedup vs baseline: 1.4808x; 1.4145x over previous
import functools

import jax
import jax.numpy as jnp
from jax import lax
from jax.experimental import pallas as pl
from jax.experimental.pallas import tpu as pltpu
from jax.experimental.pallas import tpu_sc as plsc

B, T, D = 4096, 77, 768
NW = 32
BPW = B // NW
CH = 32
NSUB = BPW // CH
NG = T * NSUB


def _body(xT, tok, pos, out, idx_all, pos_all, rowsA, rowsB,
          gsemA, gsemB, ssemA, ssemB):
    wid = lax.axis_index("s") * 2 + lax.axis_index("c")
    b0 = wid * BPW

    pltpu.sync_copy(xT.at[:, pl.ds(b0, BPW)], idx_all)
    pltpu.sync_copy(pos, pos_all)

    def idx_ref(g):
        return idx_all.at[g // NSUB, pl.ds((g % NSUB) * CH, CH)]

    def out_ref(g):
        return out.at[pl.ds(b0 + (g % NSUB) * CH, CH),
                      pl.ds(g // NSUB, 1)]

    def add_pos(g, buf):
        t = g // NSUB
        for h in range(2):
            pv = tuple(pos_all[t, pl.ds(h * 384 + j * 16, 16)]
                       for j in range(24))

            def r_body(r, carry):
                for j in range(24):
                    plsc.addupdate(buf.at[r, 0, pl.ds(h * 384 + j * 16, 16)],
                                   carry[j])
                return carry

            lax.fori_loop(0, CH, r_body, pv)

    bufs = ((rowsA, gsemA, ssemA), (rowsB, gsemB, ssemB))
    pltpu.async_copy(tok.at[idx_ref(0)], rowsA, gsemA)

    def g2_body(g2, _):
        for bpar in range(2):
            g = g2 * 2 + bpar
            cur_buf, cur_g, cur_s = bufs[bpar]
            nxt_buf, nxt_g, nxt_s = bufs[1 - bpar]

            @pl.when(g >= 1)
            def _():
                pltpu.make_async_copy(nxt_buf, out_ref(g - 1), nxt_s).wait()

            @pl.when(g + 1 < NG)
            def _():
                pltpu.async_copy(tok.at[idx_ref(g + 1)], nxt_buf, nxt_g)

            pltpu.make_async_copy(tok.at[idx_ref(g)], cur_buf, cur_g).wait()
            add_pos(g, cur_buf)
            pltpu.async_copy(cur_buf, out_ref(g), cur_s)
        return 0

    lax.fori_loop(0, NG // 2, g2_body, 0)
    pltpu.make_async_copy(rowsB, out_ref(NG - 1), ssemB).wait()


@jax.jit
def kernel(x, tok_embed, pos_embed):
    xT = x.astype(jnp.int32).T
    tok3 = tok_embed.reshape(tok_embed.shape[0], 1, D)
    mesh = plsc.VectorSubcoreMesh(core_axis_name="c", subcore_axis_name="s")
    k = functools.partial(
        pl.kernel,
        mesh=mesh,
        out_type=jax.ShapeDtypeStruct((B, T, D), jnp.float32),
        scratch_types=[
            pltpu.VMEM((T, BPW), jnp.int32),
            pltpu.VMEM((T, D), jnp.float32),
            pltpu.VMEM((CH, 1, D), jnp.float32),
            pltpu.VMEM((CH, 1, D), jnp.float32),
            pltpu.SemaphoreType.DMA,
            pltpu.SemaphoreType.DMA,
            pltpu.SemaphoreType.DMA,
            pltpu.SemaphoreType.DMA,
        ],
    )(_body)
    return k(xT, tok3, pos_embed)
